# baseline (speedup 1.0000x reference)
import functools

import jax
import jax.numpy as jnp
from jax import lax
from jax.experimental import pallas as pl
from jax.experimental.pallas import tpu as pltpu
from jax.experimental.pallas import tpu_sc as plsc

N = 10000
D = 128
E = 160000
PASSES = 3

NS = 16
K = 128
IB = 16
NBLK = 5
CHUNKS = IB * NBLK
EPT = E // NS
EPP = NBLK * IB * K
NP = 10240
RPT = NP // NS
PAD_DST = N + 200

BN = 1000
NB = N // BN


@functools.lru_cache(maxsize=None)
def _build_sc_scatter():
    mesh = plsc.VectorSubcoreMesh(core_axis_name="c", subcore_axis_name="s")

    @functools.partial(
        pl.kernel,
        out_type=jax.ShapeDtypeStruct((2, NP, D), jnp.float32),
        mesh=mesh,
        scratch_types=[
            pltpu.VMEM_SHARED((NP, D), jnp.float32),
            pltpu.VMEM((IB, K), jnp.int32),
            pltpu.VMEM((IB, K), jnp.int32),
            pltpu.VMEM((K, D), jnp.float32),
            pltpu.VMEM((K, D), jnp.float32),
            pltpu.SemaphoreType.DMA,
            pltpu.SemaphoreType.DMA,
        ],
    )
    def body(u_hbm, src_hbm, dst_hbm, z_hbm, out_hbm,
             acc, isrc, idst, rows_a, rows_b, sem_a, sem_b):
        _sc_scatter_body(u_hbm, src_hbm, dst_hbm, z_hbm, out_hbm,
                         acc, isrc, idst, rows_a, rows_b, sem_a, sem_b)

    return body


def _sc_scatter(u, src, dst, z128):
    return _build_sc_scatter()(u, src, dst, z128)


def _sc_scatter_body(u_hbm, src_hbm, dst_hbm, z_hbm, out_hbm,
                     acc, isrc, idst, rows_a, rows_b, sem_a, sem_b):
    c = lax.axis_index("c")
    s = lax.axis_index("s")
    base = s * RPT
    pltpu.sync_copy(z_hbm.at[pl.ds(base, RPT)], acc.at[pl.ds(base, RPT)])
    plsc.subcore_barrier()

    @pl.loop(0, NBLK)
    def _(b):
        pltpu.sync_copy(src_hbm.at[c].at[s].at[b], isrc)
        pltpu.sync_copy(dst_hbm.at[c].at[s].at[b], idst)

        pltpu.async_copy(u_hbm.at[c].at[isrc.at[0]], rows_a, sem_a)

        @pl.loop(0, IB, step=2)
        def _(j):
            cp_b = pltpu.async_copy(u_hbm.at[c].at[isrc.at[j + 1]], rows_b, sem_b)
            pltpu.make_async_copy(u_hbm.at[c].at[isrc.at[j]], rows_a, sem_a).wait()
            pltpu.sync_copy(rows_a, acc.at[idst.at[j]], add=True)

            @pl.when(j + 2 < IB)
            def _():
                pltpu.async_copy(u_hbm.at[c].at[isrc.at[j + 2]], rows_a, sem_a)
            cp_b.wait()
            pltpu.sync_copy(rows_b, acc.at[idst.at[j + 1]], add=True)

    plsc.subcore_barrier()
    pltpu.sync_copy(acc.at[pl.ds(base, RPT)], out_hbm.at[c].at[pl.ds(base, RPT)])


def _leaky(x):
    return jnp.where(x >= 0, x, 0.01 * x)


def _dot(a, b):
    return jax.lax.dot_general(a, b, (((1,), (0,)), ((), ())),
                               preferred_element_type=jnp.float32)


def _v_node(h, w1t_ref, b1_ref, w2t_ref, b2_ref):
    u = _leaky(_dot(h, w1t_ref[...]) + b1_ref[...])
    return _dot(u, w2t_ref[...]) + b2_ref[...]


def _v_init_body(h_ref, w10_ref, b10_ref, w20_ref, b20_ref,
                 w11_ref, b11_ref, w21_ref, b21_ref, v_ref):
    h = h_ref[...]
    v_ref[0] = _v_node(h, w10_ref, b10_ref, w20_ref, b20_ref)
    v_ref[1] = _v_node(h, w11_ref, b11_ref, w21_ref, b21_ref)


def _full(shape):
    return pl.BlockSpec(shape, lambda i: tuple(0 for _ in shape))


def _v_init(h, w1t0, b10, w2t0, b20, w1t1, b11, w2t1, b21):
    return pl.pallas_call(
        _v_init_body,
        grid=(NB,),
        in_specs=[
            pl.BlockSpec((BN, D), lambda i: (i, 0)),
            _full((D, D)), _full((1, D)), _full((D, D)), _full((1, D)),
            _full((D, D)), _full((1, D)), _full((D, D)), _full((1, D)),
        ],
        out_specs=pl.BlockSpec((2, BN, D), lambda i: (0, i, 0)),
        out_shape=jax.ShapeDtypeStruct((2, N, D), jnp.float32),
    )(h, w1t0, b10, w2t0, b20, w1t1, b11, w2t1, b21)


def _gru_block(h, inc, wit_ref, bi_ref, wht_ref, bh_ref):
    gi = _dot(inc, wit_ref[...]) + bi_ref[...]
    gh = _dot(h, wht_ref[...]) + bh_ref[...]
    r = jax.nn.sigmoid(gi[:, :D] + gh[:, :D])
    z = jax.nn.sigmoid(gi[:, D:2 * D] + gh[:, D:2 * D])
    n = jnp.tanh(gi[:, 2 * D:] + r * gh[:, 2 * D:])
    return (1.0 - z) * n + z * h


def _step_body(h_ref, s_ref,
               wit_ref, bi_ref, wht_ref, bh_ref,
               w10_ref, b10_ref, w20_ref, b20_ref,
               w11_ref, b11_ref, w21_ref, b21_ref,
               h_out_ref, v_ref):
    h = h_ref[...]
    inc = s_ref[0] + s_ref[1]
    hn = _gru_block(h, inc, wit_ref, bi_ref, wht_ref, bh_ref)
    h_out_ref[...] = hn
    v_ref[0] = _v_node(hn, w10_ref, b10_ref, w20_ref, b20_ref)
    v_ref[1] = _v_node(hn, w11_ref, b11_ref, w21_ref, b21_ref)


def _step(h, s_stack, wit, bi, wht, bh,
          w1t0, b10, w2t0, b20, w1t1, b11, w2t1, b21):
    return pl.pallas_call(
        _step_body,
        grid=(NB,),
        in_specs=[
            pl.BlockSpec((BN, D), lambda i: (i, 0)),
            pl.BlockSpec((2, BN, D), lambda i: (0, i, 0)),
            _full((D, 3 * D)), _full((1, 3 * D)), _full((D, 3 * D)), _full((1, 3 * D)),
            _full((D, D)), _full((1, D)), _full((D, D)), _full((1, D)),
            _full((D, D)), _full((1, D)), _full((D, D)), _full((1, D)),
        ],
        out_specs=[
            pl.BlockSpec((BN, D), lambda i: (i, 0)),
            pl.BlockSpec((2, BN, D), lambda i: (0, i, 0)),
        ],
        out_shape=[
            jax.ShapeDtypeStruct((N, D), jnp.float32),
            jax.ShapeDtypeStruct((2, N, D), jnp.float32),
        ],
    )(h, s_stack, wit, bi, wht, bh,
      w1t0, b10, w2t0, b20, w1t1, b11, w2t1, b21)


def _last_body(h_ref, s_ref,
               wit_ref, bi_ref, wht_ref, bh_ref,
               pt_ref, f1g_ref, f1p_ref, f1b_ref, f2_ref, f2b_ref,
               fl_ref, flb_ref,
               out_ref, gsum):
    i = pl.program_id(0)
    h = h_ref[...]
    inc = s_ref[0] + s_ref[1]
    hn = _gru_block(h, inc, wit_ref, bi_ref, wht_ref, bh_ref)
    part = jnp.sum(hn, axis=0, keepdims=True)

    @pl.when(i == 0)
    def _():
        gsum[...] = part

    @pl.when(i > 0)
    def _():
        gsum[...] = gsum[...] + part

    @pl.when(i == NB - 1)
    def _():
        g = gsum[...]
        g = jnp.log(g)
        g = jnp.where(jnp.isnan(g), 0.0, g)
        g = jnp.maximum(g, 0.0)
        isinf = jnp.isinf(g)
        finite_max = jnp.max(jnp.where(isinf, -jnp.inf, g))
        g = jnp.where(isinf, finite_max, g)
        x = _dot(g, f1g_ref[...]) + pt_ref[...] * f1p_ref[...] + f1b_ref[...]
        x = _leaky(x)
        x = _leaky(_dot(x, f2_ref[...]) + f2b_ref[...])
        out_ref[...] = _dot(x, fl_ref[...]) + flb_ref[...]


def _last(h, s_stack, wit, bi, wht, bh,
          pt, f1g, f1p, f1b, f2, f2b, fl, flb):
    return pl.pallas_call(
        _last_body,
        grid=(NB,),
        in_specs=[
            pl.BlockSpec((BN, D), lambda i: (i, 0)),
            pl.BlockSpec((2, BN, D), lambda i: (0, i, 0)),
            _full((D, 3 * D)), _full((1, 3 * D)), _full((D, 3 * D)), _full((1, 3 * D)),
            _full((1, 1)), _full((D, 80)), _full((1, 80)), _full((1, 80)),
            _full((80, 80)), _full((1, 80)), _full((80, 2)), _full((1, 2)),
        ],
        out_specs=pl.BlockSpec((1, 2), lambda i: (0, 0)),
        out_shape=jax.ShapeDtypeStruct((1, 2), jnp.float32),
        scratch_shapes=[pltpu.VMEM((1, D), jnp.float32)],
    )(h, s_stack, wit, bi, wht, bh,
      pt, f1g, f1p, f1b, f2, f2b, fl, flb)


def kernel(nodes, edge_set_0, edge_set_1, problem_type,
           W1_0, b1_0, W2_0, b2_0, W1_1, b1_1, W2_1, b2_1,
           gru_wi, gru_wh, gru_bi, gru_bh,
           fc1_w, fc1_b, fc2_w, fc2_b, fcl_w, fcl_b):
    def _lay(col, fill):
        e0 = jnp.pad(edge_set_0[:, col].reshape(NS, EPT),
                     ((0, 0), (0, EPP - EPT)), constant_values=fill)
        e1 = jnp.pad(edge_set_1[:, col].reshape(NS, EPT),
                     ((0, 0), (0, EPP - EPT)), constant_values=fill)
        return jnp.stack([e0.reshape(NS, NBLK, IB, K),
                          e1.reshape(NS, NBLK, IB, K)])

    src = _lay(1, 0)
    dst = _lay(0, PAD_DST)
    z128 = jnp.zeros((NP, D), jnp.float32)

    w1t0, w1t1 = W1_0.T, W1_1.T
    w2t0, w2t1 = W2_0.T, W2_1.T
    b10, b11 = b1_0[None, :], b1_1[None, :]
    b20, b21 = b2_0[None, :], b2_1[None, :]
    wit, wht = gru_wi.T, gru_wh.T
    bi, bh = gru_bi[None, :], gru_bh[None, :]
    f1g = fc1_w[:, :D].T
    f1p = fc1_w[:, D:D + 1].T
    f1b = fc1_b[None, :]
    f2, f2b = fc2_w.T, fc2_b[None, :]
    fl, flb = fcl_w.T, fcl_b[None, :]

    h = nodes
    v = _v_init(h, w1t0, b10, w2t0, b20, w1t1, b11, w2t1, b21)
    for p in range(PASSES):
        s_stack = _sc_scatter(v, src, dst, z128)
        if p < PASSES - 1:
            h, v = _step(h, s_stack, wit, bi, wht, bh,
                         w1t0, b10, w2t0, b20, w1t1, b11, w2t1, b21)
        else:
            out = _last(h, s_stack, wit, bi, wht, bh, problem_type,
                        f1g, f1p, f1b, f2, f2b, fl, flb)
    return out


# speedup vs baseline: 1.8488x; 1.8488x over previous
import functools

import jax
import jax.numpy as jnp
from jax import lax
from jax.experimental import pallas as pl
from jax.experimental.pallas import tpu as pltpu
from jax.experimental.pallas import tpu_sc as plsc

N = 10000
D = 128
E = 160000
PASSES = 3

NS = 16
K = 100
IB = 20
NBLK = 5
CHUNKS = IB * NBLK
EPT = E // NS
NP = 10240
RPT = NP // NS

BN = 1000
NB = N // BN


@functools.lru_cache(maxsize=None)
def _build_sc_scatter():
    mesh = plsc.VectorSubcoreMesh(core_axis_name="c", subcore_axis_name="s")

    @functools.partial(
        pl.kernel,
        out_type=jax.ShapeDtypeStruct((2, NP, D), jnp.float32),
        mesh=mesh,
        scratch_types=[
            pltpu.VMEM_SHARED((NP, D), jnp.float32),
            pltpu.VMEM((IB, K), jnp.int32),
            pltpu.VMEM((IB, K), jnp.int32),
            pltpu.VMEM((K, D), jnp.float32),
            pltpu.VMEM((K, D), jnp.float32),
            pltpu.SemaphoreType.DMA,
            pltpu.SemaphoreType.DMA,
        ],
    )
    def body(u_hbm, src_hbm, dst_hbm, z_hbm, out_hbm,
             acc, isrc, idst, rows_a, rows_b, sem_a, sem_b):
        _sc_scatter_body(u_hbm, src_hbm, dst_hbm, z_hbm, out_hbm,
                         acc, isrc, idst, rows_a, rows_b, sem_a, sem_b)

    return body


def _sc_scatter(u, src, dst, z128):
    return _build_sc_scatter()(u, src, dst, z128)


def _sc_scatter_body(u_hbm, src_hbm, dst_hbm, z_hbm, out_hbm,
                     acc, isrc, idst, rows_a, rows_b, sem_a, sem_b):
    c = lax.axis_index("c")
    s = lax.axis_index("s")
    base = s * RPT
    pltpu.sync_copy(z_hbm.at[pl.ds(base, RPT)], acc.at[pl.ds(base, RPT)])
    plsc.subcore_barrier()

    @pl.loop(0, NBLK)
    def _(b):
        pltpu.sync_copy(src_hbm.at[c].at[s].at[b], isrc)
        pltpu.sync_copy(dst_hbm.at[c].at[s].at[b], idst)

        pltpu.async_copy(u_hbm.at[c].at[isrc.at[0]], rows_a, sem_a)

        @pl.loop(0, IB, step=2)
        def _(j):
            cp_b = pltpu.async_copy(u_hbm.at[c].at[isrc.at[j + 1]], rows_b, sem_b)
            pltpu.make_async_copy(u_hbm.at[c].at[isrc.at[j]], rows_a, sem_a).wait()
            pltpu.sync_copy(rows_a, acc.at[idst.at[j]], add=True)

            @pl.when(j + 2 < IB)
            def _():
                pltpu.async_copy(u_hbm.at[c].at[isrc.at[j + 2]], rows_a, sem_a)
            cp_b.wait()
            pltpu.sync_copy(rows_b, acc.at[idst.at[j + 1]], add=True)

    plsc.subcore_barrier()
    pltpu.sync_copy(acc.at[pl.ds(base, RPT)], out_hbm.at[c].at[pl.ds(base, RPT)])


def _leaky(x):
    return jnp.where(x >= 0, x, 0.01 * x)


def _dot(a, b):
    return jax.lax.dot_general(a, b, (((1,), (0,)), ((), ())),
                               preferred_element_type=jnp.float32)


def _v_node(h, w1t_ref, b1_ref, w2t_ref, b2_ref):
    u = _leaky(_dot(h, w1t_ref[...]) + b1_ref[...])
    return _dot(u, w2t_ref[...]) + b2_ref[...]


def _v_init_body(h_ref, w10_ref, b10_ref, w20_ref, b20_ref,
                 w11_ref, b11_ref, w21_ref, b21_ref, v_ref):
    h = h_ref[...]
    v_ref[0] = _v_node(h, w10_ref, b10_ref, w20_ref, b20_ref)
    v_ref[1] = _v_node(h, w11_ref, b11_ref, w21_ref, b21_ref)


def _full(shape):
    return pl.BlockSpec(shape, lambda i: tuple(0 for _ in shape))


def _v_init(h, w1t0, b10, w2t0, b20, w1t1, b11, w2t1, b21):
    return pl.pallas_call(
        _v_init_body,
        grid=(NB,),
        in_specs=[
            pl.BlockSpec((BN, D), lambda i: (i, 0)),
            _full((D, D)), _full((1, D)), _full((D, D)), _full((1, D)),
            _full((D, D)), _full((1, D)), _full((D, D)), _full((1, D)),
        ],
        out_specs=pl.BlockSpec((2, BN, D), lambda i: (0, i, 0)),
        out_shape=jax.ShapeDtypeStruct((2, N, D), jnp.float32),
    )(h, w1t0, b10, w2t0, b20, w1t1, b11, w2t1, b21)


def _gru_block(h, inc, wit_ref, bi_ref, wht_ref, bh_ref):
    gi = _dot(inc, wit_ref[...]) + bi_ref[...]
    gh = _dot(h, wht_ref[...]) + bh_ref[...]
    r = jax.nn.sigmoid(gi[:, :D] + gh[:, :D])
    z = jax.nn.sigmoid(gi[:, D:2 * D] + gh[:, D:2 * D])
    n = jnp.tanh(gi[:, 2 * D:] + r * gh[:, 2 * D:])
    return (1.0 - z) * n + z * h


def _step_body(h_ref, s_ref,
               wit_ref, bi_ref, wht_ref, bh_ref,
               w10_ref, b10_ref, w20_ref, b20_ref,
               w11_ref, b11_ref, w21_ref, b21_ref,
               h_out_ref, v_ref):
    h = h_ref[...]
    inc = s_ref[0] + s_ref[1]
    hn = _gru_block(h, inc, wit_ref, bi_ref, wht_ref, bh_ref)
    h_out_ref[...] = hn
    v_ref[0] = _v_node(hn, w10_ref, b10_ref, w20_ref, b20_ref)
    v_ref[1] = _v_node(hn, w11_ref, b11_ref, w21_ref, b21_ref)


def _step(h, s_stack, wit, bi, wht, bh,
          w1t0, b10, w2t0, b20, w1t1, b11, w2t1, b21):
    return pl.pallas_call(
        _step_body,
        grid=(NB,),
        in_specs=[
            pl.BlockSpec((BN, D), lambda i: (i, 0)),
            pl.BlockSpec((2, BN, D), lambda i: (0, i, 0)),
            _full((D, 3 * D)), _full((1, 3 * D)), _full((D, 3 * D)), _full((1, 3 * D)),
            _full((D, D)), _full((1, D)), _full((D, D)), _full((1, D)),
            _full((D, D)), _full((1, D)), _full((D, D)), _full((1, D)),
        ],
        out_specs=[
            pl.BlockSpec((BN, D), lambda i: (i, 0)),
            pl.BlockSpec((2, BN, D), lambda i: (0, i, 0)),
        ],
        out_shape=[
            jax.ShapeDtypeStruct((N, D), jnp.float32),
            jax.ShapeDtypeStruct((2, N, D), jnp.float32),
        ],
    )(h, s_stack, wit, bi, wht, bh,
      w1t0, b10, w2t0, b20, w1t1, b11, w2t1, b21)


def _last_body(h_ref, s_ref,
               wit_ref, bi_ref, wht_ref, bh_ref,
               pt_ref, f1g_ref, f1p_ref, f1b_ref, f2_ref, f2b_ref,
               fl_ref, flb_ref,
               out_ref, gsum):
    i = pl.program_id(0)
    h = h_ref[...]
    inc = s_ref[0] + s_ref[1]
    hn = _gru_block(h, inc, wit_ref, bi_ref, wht_ref, bh_ref)
    part = jnp.sum(hn, axis=0, keepdims=True)

    @pl.when(i == 0)
    def _():
        gsum[...] = part

    @pl.when(i > 0)
    def _():
        gsum[...] = gsum[...] + part

    @pl.when(i == NB - 1)
    def _():
        g = gsum[...]
        g = jnp.log(g)
        g = jnp.where(jnp.isnan(g), 0.0, g)
        g = jnp.maximum(g, 0.0)
        isinf = jnp.isinf(g)
        finite_max = jnp.max(jnp.where(isinf, -jnp.inf, g))
        g = jnp.where(isinf, finite_max, g)
        x = _dot(g, f1g_ref[...]) + pt_ref[...] * f1p_ref[...] + f1b_ref[...]
        x = _leaky(x)
        x = _leaky(_dot(x, f2_ref[...]) + f2b_ref[...])
        out_ref[...] = _dot(x, fl_ref[...]) + flb_ref[...]


def _last(h, s_stack, wit, bi, wht, bh,
          pt, f1g, f1p, f1b, f2, f2b, fl, flb):
    return pl.pallas_call(
        _last_body,
        grid=(NB,),
        in_specs=[
            pl.BlockSpec((BN, D), lambda i: (i, 0)),
            pl.BlockSpec((2, BN, D), lambda i: (0, i, 0)),
            _full((D, 3 * D)), _full((1, 3 * D)), _full((D, 3 * D)), _full((1, 3 * D)),
            _full((1, 1)), _full((D, 80)), _full((1, 80)), _full((1, 80)),
            _full((80, 80)), _full((1, 80)), _full((80, 2)), _full((1, 2)),
        ],
        out_specs=pl.BlockSpec((1, 2), lambda i: (0, 0)),
        out_shape=jax.ShapeDtypeStruct((1, 2), jnp.float32),
        scratch_shapes=[pltpu.VMEM((1, D), jnp.float32)],
    )(h, s_stack, wit, bi, wht, bh,
      pt, f1g, f1p, f1b, f2, f2b, fl, flb)


def kernel(nodes, edge_set_0, edge_set_1, problem_type,
           W1_0, b1_0, W2_0, b2_0, W1_1, b1_1, W2_1, b2_1,
           gru_wi, gru_wh, gru_bi, gru_bh,
           fc1_w, fc1_b, fc2_w, fc2_b, fcl_w, fcl_b):
    src = jnp.stack([edge_set_0[:, 1].reshape(NS, NBLK, IB, K),
                     edge_set_1[:, 1].reshape(NS, NBLK, IB, K)])
    dst = jnp.stack([edge_set_0[:, 0].reshape(NS, NBLK, IB, K),
                     edge_set_1[:, 0].reshape(NS, NBLK, IB, K)])
    z128 = jnp.zeros((NP, D), jnp.float32)

    w1t0, w1t1 = W1_0.T, W1_1.T
    w2t0, w2t1 = W2_0.T, W2_1.T
    b10, b11 = b1_0[None, :], b1_1[None, :]
    b20, b21 = b2_0[None, :], b2_1[None, :]
    wit, wht = gru_wi.T, gru_wh.T
    bi, bh = gru_bi[None, :], gru_bh[None, :]
    f1g = fc1_w[:, :D].T
    f1p = fc1_w[:, D:D + 1].T
    f1b = fc1_b[None, :]
    f2, f2b = fc2_w.T, fc2_b[None, :]
    fl, flb = fcl_w.T, fcl_b[None, :]

    h = nodes
    v = _v_init(h, w1t0, b10, w2t0, b20, w1t1, b11, w2t1, b21)
    for p in range(PASSES):
        s_stack = _sc_scatter(v, src, dst, z128)
        if p < PASSES - 1:
            h, v = _step(h, s_stack, wit, bi, wht, bh,
                         w1t0, b10, w2t0, b20, w1t1, b11, w2t1, b21)
        else:
            out = _last(h, s_stack, wit, bi, wht, bh, problem_type,
                        f1g, f1p, f1b, f2, f2b, fl, flb)
    return out


# speedup vs baseline: 1.9487x; 1.0540x over previous
import functools

import jax
import jax.numpy as jnp
from jax import lax
from jax.experimental import pallas as pl
from jax.experimental.pallas import tpu as pltpu
from jax.experimental.pallas import tpu_sc as plsc

N = 10000
D = 128
E = 160000
PASSES = 3

NS = 16
K = 100
IB = 50
NBLK = 2
CHUNKS = IB * NBLK
EPT = E // NS
NP = 10240
RPT = NP // NS

BN = 1000
NB = N // BN


@functools.lru_cache(maxsize=None)
def _build_sc_scatter():
    mesh = plsc.VectorSubcoreMesh(core_axis_name="c", subcore_axis_name="s")

    @functools.partial(
        pl.kernel,
        out_type=jax.ShapeDtypeStruct((2, NP, D), jnp.float32),
        mesh=mesh,
        scratch_types=[
            pltpu.VMEM_SHARED((NP, D), jnp.float32),
            pltpu.VMEM((IB, K), jnp.int32),
            pltpu.VMEM((IB, K), jnp.int32),
            pltpu.VMEM((K, D), jnp.float32),
            pltpu.VMEM((K, D), jnp.float32),
            pltpu.SemaphoreType.DMA,
            pltpu.SemaphoreType.DMA,
        ],
    )
    def body(u_hbm, src_hbm, dst_hbm, z_hbm, out_hbm,
             acc, isrc, idst, rows_a, rows_b, sem_a, sem_b):
        _sc_scatter_body(u_hbm, src_hbm, dst_hbm, z_hbm, out_hbm,
                         acc, isrc, idst, rows_a, rows_b, sem_a, sem_b)

    return body


def _sc_scatter(u, src, dst, z128):
    return _build_sc_scatter()(u, src, dst, z128)


def _sc_scatter_body(u_hbm, src_hbm, dst_hbm, z_hbm, out_hbm,
                     acc, isrc, idst, rows_a, rows_b, sem_a, sem_b):
    c = lax.axis_index("c")
    s = lax.axis_index("s")
    base = s * RPT
    pltpu.sync_copy(z_hbm.at[pl.ds(base, RPT)], acc.at[pl.ds(base, RPT)])
    plsc.subcore_barrier()

    @pl.loop(0, NBLK)
    def _(b):
        pltpu.sync_copy(src_hbm.at[c].at[s].at[b], isrc)
        pltpu.sync_copy(dst_hbm.at[c].at[s].at[b], idst)

        pltpu.async_copy(u_hbm.at[c].at[isrc.at[0]], rows_a, sem_a)

        @pl.loop(0, IB, step=2)
        def _(j):
            cp_b = pltpu.async_copy(u_hbm.at[c].at[isrc.at[j + 1]], rows_b, sem_b)
            pltpu.make_async_copy(u_hbm.at[c].at[isrc.at[j]], rows_a, sem_a).wait()
            pltpu.sync_copy(rows_a, acc.at[idst.at[j]], add=True)

            @pl.when(j + 2 < IB)
            def _():
                pltpu.async_copy(u_hbm.at[c].at[isrc.at[j + 2]], rows_a, sem_a)
            cp_b.wait()
            pltpu.sync_copy(rows_b, acc.at[idst.at[j + 1]], add=True)

    plsc.subcore_barrier()
    pltpu.sync_copy(acc.at[pl.ds(base, RPT)], out_hbm.at[c].at[pl.ds(base, RPT)])


def _leaky(x):
    return jnp.where(x >= 0, x, 0.01 * x)


def _dot(a, b):
    return jax.lax.dot_general(a, b, (((1,), (0,)), ((), ())),
                               preferred_element_type=jnp.float32)


def _v_node(h, w1t_ref, b1_ref, w2t_ref, b2_ref):
    u = _leaky(_dot(h, w1t_ref[...]) + b1_ref[...])
    return _dot(u, w2t_ref[...]) + b2_ref[...]


def _v_init_body(h_ref, w10_ref, b10_ref, w20_ref, b20_ref,
                 w11_ref, b11_ref, w21_ref, b21_ref, v_ref):
    h = h_ref[...]
    v_ref[0] = _v_node(h, w10_ref, b10_ref, w20_ref, b20_ref)
    v_ref[1] = _v_node(h, w11_ref, b11_ref, w21_ref, b21_ref)


def _full(shape):
    return pl.BlockSpec(shape, lambda i: tuple(0 for _ in shape))


def _v_init(h, w1t0, b10, w2t0, b20, w1t1, b11, w2t1, b21):
    return pl.pallas_call(
        _v_init_body,
        grid=(NB,),
        in_specs=[
            pl.BlockSpec((BN, D), lambda i: (i, 0)),
            _full((D, D)), _full((1, D)), _full((D, D)), _full((1, D)),
            _full((D, D)), _full((1, D)), _full((D, D)), _full((1, D)),
        ],
        out_specs=pl.BlockSpec((2, BN, D), lambda i: (0, i, 0)),
        out_shape=jax.ShapeDtypeStruct((2, N, D), jnp.float32),
    )(h, w1t0, b10, w2t0, b20, w1t1, b11, w2t1, b21)


def _gru_block(h, inc, wit_ref, bi_ref, wht_ref, bh_ref):
    gi = _dot(inc, wit_ref[...]) + bi_ref[...]
    gh = _dot(h, wht_ref[...]) + bh_ref[...]
    r = jax.nn.sigmoid(gi[:, :D] + gh[:, :D])
    z = jax.nn.sigmoid(gi[:, D:2 * D] + gh[:, D:2 * D])
    n = jnp.tanh(gi[:, 2 * D:] + r * gh[:, 2 * D:])
    return (1.0 - z) * n + z * h


def _step_body(h_ref, s_ref,
               wit_ref, bi_ref, wht_ref, bh_ref,
               w10_ref, b10_ref, w20_ref, b20_ref,
               w11_ref, b11_ref, w21_ref, b21_ref,
               h_out_ref, v_ref):
    h = h_ref[...]
    inc = s_ref[0] + s_ref[1]
    hn = _gru_block(h, inc, wit_ref, bi_ref, wht_ref, bh_ref)
    h_out_ref[...] = hn
    v_ref[0] = _v_node(hn, w10_ref, b10_ref, w20_ref, b20_ref)
    v_ref[1] = _v_node(hn, w11_ref, b11_ref, w21_ref, b21_ref)


def _step(h, s_stack, wit, bi, wht, bh,
          w1t0, b10, w2t0, b20, w1t1, b11, w2t1, b21):
    return pl.pallas_call(
        _step_body,
        grid=(NB,),
        in_specs=[
            pl.BlockSpec((BN, D), lambda i: (i, 0)),
            pl.BlockSpec((2, BN, D), lambda i: (0, i, 0)),
            _full((D, 3 * D)), _full((1, 3 * D)), _full((D, 3 * D)), _full((1, 3 * D)),
            _full((D, D)), _full((1, D)), _full((D, D)), _full((1, D)),
            _full((D, D)), _full((1, D)), _full((D, D)), _full((1, D)),
        ],
        out_specs=[
            pl.BlockSpec((BN, D), lambda i: (i, 0)),
            pl.BlockSpec((2, BN, D), lambda i: (0, i, 0)),
        ],
        out_shape=[
            jax.ShapeDtypeStruct((N, D), jnp.float32),
            jax.ShapeDtypeStruct((2, N, D), jnp.float32),
        ],
    )(h, s_stack, wit, bi, wht, bh,
      w1t0, b10, w2t0, b20, w1t1, b11, w2t1, b21)


def _last_body(h_ref, s_ref,
               wit_ref, bi_ref, wht_ref, bh_ref,
               pt_ref, f1g_ref, f1p_ref, f1b_ref, f2_ref, f2b_ref,
               fl_ref, flb_ref,
               out_ref, gsum):
    i = pl.program_id(0)
    h = h_ref[...]
    inc = s_ref[0] + s_ref[1]
    hn = _gru_block(h, inc, wit_ref, bi_ref, wht_ref, bh_ref)
    part = jnp.sum(hn, axis=0, keepdims=True)

    @pl.when(i == 0)
    def _():
        gsum[...] = part

    @pl.when(i > 0)
    def _():
        gsum[...] = gsum[...] + part

    @pl.when(i == NB - 1)
    def _():
        g = gsum[...]
        g = jnp.log(g)
        g = jnp.where(jnp.isnan(g), 0.0, g)
        g = jnp.maximum(g, 0.0)
        isinf = jnp.isinf(g)
        finite_max = jnp.max(jnp.where(isinf, -jnp.inf, g))
        g = jnp.where(isinf, finite_max, g)
        x = _dot(g, f1g_ref[...]) + pt_ref[...] * f1p_ref[...] + f1b_ref[...]
        x = _leaky(x)
        x = _leaky(_dot(x, f2_ref[...]) + f2b_ref[...])
        out_ref[...] = _dot(x, fl_ref[...]) + flb_ref[...]


def _last(h, s_stack, wit, bi, wht, bh,
          pt, f1g, f1p, f1b, f2, f2b, fl, flb):
    return pl.pallas_call(
        _last_body,
        grid=(NB,),
        in_specs=[
            pl.BlockSpec((BN, D), lambda i: (i, 0)),
            pl.BlockSpec((2, BN, D), lambda i: (0, i, 0)),
            _full((D, 3 * D)), _full((1, 3 * D)), _full((D, 3 * D)), _full((1, 3 * D)),
            _full((1, 1)), _full((D, 80)), _full((1, 80)), _full((1, 80)),
            _full((80, 80)), _full((1, 80)), _full((80, 2)), _full((1, 2)),
        ],
        out_specs=pl.BlockSpec((1, 2), lambda i: (0, 0)),
        out_shape=jax.ShapeDtypeStruct((1, 2), jnp.float32),
        scratch_shapes=[pltpu.VMEM((1, D), jnp.float32)],
    )(h, s_stack, wit, bi, wht, bh,
      pt, f1g, f1p, f1b, f2, f2b, fl, flb)


def kernel(nodes, edge_set_0, edge_set_1, problem_type,
           W1_0, b1_0, W2_0, b2_0, W1_1, b1_1, W2_1, b2_1,
           gru_wi, gru_wh, gru_bi, gru_bh,
           fc1_w, fc1_b, fc2_w, fc2_b, fcl_w, fcl_b):
    src = jnp.stack([edge_set_0[:, 1].reshape(NS, NBLK, IB, K),
                     edge_set_1[:, 1].reshape(NS, NBLK, IB, K)])
    dst = jnp.stack([edge_set_0[:, 0].reshape(NS, NBLK, IB, K),
                     edge_set_1[:, 0].reshape(NS, NBLK, IB, K)])
    z128 = jnp.zeros((NP, D), jnp.float32)

    w1t0, w1t1 = W1_0.T, W1_1.T
    w2t0, w2t1 = W2_0.T, W2_1.T
    b10, b11 = b1_0[None, :], b1_1[None, :]
    b20, b21 = b2_0[None, :], b2_1[None, :]
    wit, wht = gru_wi.T, gru_wh.T
    bi, bh = gru_bi[None, :], gru_bh[None, :]
    f1g = fc1_w[:, :D].T
    f1p = fc1_w[:, D:D + 1].T
    f1b = fc1_b[None, :]
    f2, f2b = fc2_w.T, fc2_b[None, :]
    fl, flb = fcl_w.T, fcl_b[None, :]

    h = nodes
    v = _v_init(h, w1t0, b10, w2t0, b20, w1t1, b11, w2t1, b21)
    for p in range(PASSES):
        s_stack = _sc_scatter(v, src, dst, z128)
        if p < PASSES - 1:
            h, v = _step(h, s_stack, wit, bi, wht, bh,
                         w1t0, b10, w2t0, b20, w1t1, b11, w2t1, b21)
        else:
            out = _last(h, s_stack, wit, bi, wht, bh, problem_type,
                        f1g, f1p, f1b, f2, f2b, fl, flb)
    return out
